# baseline (speedup 1.0000x reference)
import jax
import jax.numpy as jnp
from jax import lax
from jax.experimental import pallas as pl
from jax.experimental.pallas import tpu as pltpu
from jax.experimental.pallas import tpu_sc as plsc

B = 4096
L = 50
LP = 56
EMB = 128
LANES = 16
NJ = EMB // LANES

NC = 2
NS = 16
NW = NC * NS
RPW = B // NW

G = 2
NBUF = 4
CPW = RPW // G


def _body(x_hbm, lens_hbm, table_hbm, out_hbm,
          idx_v, lens_v, rows_bufs, out_v, sems):
    wid = lax.axis_index("s") * NC + lax.axis_index("c")
    base = wid * RPW

    pltpu.sync_copy(x_hbm.at[pl.ds(wid * CPW, CPW)], idx_v)
    pltpu.sync_copy(lens_hbm.at[pl.ds(base, RPW)], lens_v)

    for c in range(NBUF):
        pltpu.async_copy(table_hbm.at[idx_v.at[c]], rows_bufs[c], sems[c])

    def group(gg, carry):
        lens16 = lens_v[pl.ds(gg * LANES, LANES)]
        for c8 in range(LANES // G):
            c = gg * (LANES // G) + c8
            b = c8 % NBUF
            rows_b = rows_bufs[b]
            sem_b = sems[b]
            pltpu.make_async_copy(
                table_hbm.at[pl.ds(0, G * LP)], rows_b, sem_b).wait()

            for which in range(G):
                r = G * c + which
                len_r = lens16[G * c8 + which]

                def acc_step(l, acc, rows_b=rows_b, which=which):
                    return tuple(
                        acc[j] + rows_b[which * LP + l,
                                        pl.ds(LANES * j, LANES)]
                        for j in range(NJ))

                zeros = tuple(jnp.zeros((LANES,), jnp.float32)
                              for _ in range(NJ))
                acc = lax.fori_loop(0, len_r, acc_step, zeros)

                len_f = jnp.full((LANES,), len_r.astype(jnp.float32))
                scale = jnp.where(
                    len_r > 0, jnp.full((LANES,), 1.0) / len_f,
                    jnp.zeros((LANES,)))
                for j in range(NJ):
                    out_v[r, pl.ds(LANES * j, LANES)] = acc[j] * scale

            @pl.when(c + NBUF < CPW)
            def _(rows_b=rows_b, sem_b=sem_b, c=c):
                pltpu.async_copy(
                    table_hbm.at[idx_v.at[c + NBUF]], rows_b, sem_b)
        return carry

    lax.fori_loop(0, RPW // LANES, group, 0)

    pltpu.sync_copy(out_v, out_hbm.at[pl.ds(base, RPW)])


@jax.jit
def kernel(x, x_lens, table):
    xp = jnp.pad(x.astype(jnp.int32), ((0, 0), (0, LP - L)))
    xp = xp.reshape(B // G, G * LP)
    lens = x_lens.astype(jnp.int32)

    mesh = plsc.VectorSubcoreMesh(
        core_axis_name="c", subcore_axis_name="s",
        num_cores=NC, num_subcores=NS)
    f = pl.kernel(
        lambda x_hbm, lens_hbm, table_hbm, out_hbm,
        idx_v, lens_v, r0, r1, r2, r3, out_v, s0, s1, s2, s3:
        _body(x_hbm, lens_hbm, table_hbm, out_hbm,
              idx_v, lens_v, (r0, r1, r2, r3), out_v, (s0, s1, s2, s3)),
        out_type=jax.ShapeDtypeStruct((B, EMB), jnp.float32),
        mesh=mesh,
        scratch_types=[
            pltpu.VMEM((CPW, G * LP), jnp.int32),
            pltpu.VMEM((RPW,), jnp.int32),
            pltpu.VMEM((G * LP, EMB), jnp.float32),
            pltpu.VMEM((G * LP, EMB), jnp.float32),
            pltpu.VMEM((G * LP, EMB), jnp.float32),
            pltpu.VMEM((G * LP, EMB), jnp.float32),
            pltpu.VMEM((RPW, EMB), jnp.float32),
            pltpu.SemaphoreType.DMA,
            pltpu.SemaphoreType.DMA,
            pltpu.SemaphoreType.DMA,
            pltpu.SemaphoreType.DMA,
        ],
    )
    return f(xp, lens, table)
